# baseline (speedup 1.0000x reference)
import functools

import jax
import jax.numpy as jnp
from jax import lax
from jax.experimental import pallas as pl
from jax.experimental.pallas import tpu as pltpu
from jax.experimental.pallas import tpu_sc as plsc

D_MODEL = 32
D_VOCAB = 1000000
BATCH = 16384
HIST = 20

NUM_IDX = BATCH * HIST

NC = 2
NS = 16
NW = NC * NS

B_PER_W = NUM_IDX // NW
N_CHUNK = 8
B_CHUNK = B_PER_W // N_CHUNK


_VC = 8192
_G = _VC // 4
_NB = (D_VOCAB + _VC - 1) // _VC
V_PAD = _NB * _VC


def _repack_body(w_ref, out_ref, tmp_ref):
    tmp_ref[...] = w_ref[...].T.reshape(_G, 4, D_MODEL)
    out_ref[...] = tmp_ref[...].reshape(_G, 128)


def _repack(w):
    return pl.pallas_call(
        _repack_body,
        grid=(_NB,),
        in_specs=[pl.BlockSpec((D_MODEL, _VC), lambda i: (0, i))],
        out_specs=pl.BlockSpec((_G, 128), lambda i: (i, 0)),
        out_shape=jax.ShapeDtypeStruct((_NB * _G, 128), jnp.float32),
        scratch_shapes=[pltpu.VMEM((_G, 4, D_MODEL), jnp.float32)],
    )(w)


@functools.partial(
    pl.kernel,
    mesh=plsc.VectorSubcoreMesh(core_axis_name="c", subcore_axis_name="s"),
    out_type=jax.ShapeDtypeStruct((NUM_IDX, D_MODEL), jnp.float32),
    scratch_types=[
        pltpu.VMEM((N_CHUNK, B_CHUNK), jnp.int32),
        pltpu.VMEM((2, B_CHUNK, D_MODEL), jnp.float32),
        pltpu.SemaphoreType.DMA,
        pltpu.SemaphoreType.DMA,
    ],
    compiler_params=pltpu.CompilerParams(use_tc_tiling_on_sc=False),
)
def _gather(table_hbm, idx_hbm, out_hbm, idx_v, rows_v, gsem, wsem):
    wid = lax.axis_index("s") * NC + lax.axis_index("c")
    base = wid * B_PER_W
    pltpu.sync_copy(idx_hbm.at[wid], idx_v)
    g_prev = pltpu.async_copy(table_hbm.at[idx_v.at[0]], rows_v.at[0], gsem)
    for j in range(N_CHUNK):
        par = j % 2
        if j + 1 < N_CHUNK:
            g_next = pltpu.async_copy(
                table_hbm.at[idx_v.at[j + 1]], rows_v.at[1 - par], gsem
            )
        g_prev.wait()
        pltpu.async_copy(
            rows_v.at[par],
            out_hbm.at[pl.ds(base + j * B_CHUNK, B_CHUNK)],
            wsem,
        ).wait()
        if j + 1 < N_CHUNK:
            g_prev = g_next


def kernel(x, W_E):
    idx = x.reshape(NW, N_CHUNK, B_CHUNK)
    table = _repack(W_E).reshape(V_PAD, D_MODEL)
    out = _gather(table, idx)
    return out.reshape(BATCH, HIST, D_MODEL)


# speedup vs baseline: 1.4289x; 1.4289x over previous
import functools

import jax
import jax.numpy as jnp
from jax import lax
from jax.experimental import pallas as pl
from jax.experimental.pallas import tpu as pltpu
from jax.experimental.pallas import tpu_sc as plsc

D_MODEL = 32
D_VOCAB = 1000000
BATCH = 16384
HIST = 20

NUM_IDX = BATCH * HIST

NC = 2
NS = 16
NW = NC * NS

B_PER_W = NUM_IDX // NW
N_CHUNK = 8
B_CHUNK = B_PER_W // N_CHUNK

LANES = 16


_VC = 2048
Q = 1 << 18
_NB = Q // _VC


def _stack_t_body(w0, w1, w2, w3, out_ref):
    a = jnp.concatenate([w0[...], w1[...], w2[...], w3[...]], axis=0)
    out_ref[...] = a.T


def _stack_t(w):
    qb = Q // _VC
    last = (D_VOCAB - 1) // _VC
    specs = [
        pl.BlockSpec(
            (D_MODEL, _VC),
            functools.partial(
                lambda q, i: (0, jnp.minimum(q * qb + i, last)), q
            ),
        )
        for q in range(4)
    ]
    return pl.pallas_call(
        _stack_t_body,
        grid=(_NB,),
        in_specs=specs,
        out_specs=pl.BlockSpec((_VC, 128), lambda i: (i, 0)),
        out_shape=jax.ShapeDtypeStruct((Q, 128), jnp.float32),
    )(w, w, w, w)


@functools.partial(
    pl.kernel,
    mesh=plsc.VectorSubcoreMesh(core_axis_name="c", subcore_axis_name="s"),
    out_type=jax.ShapeDtypeStruct((NUM_IDX, D_MODEL), jnp.float32),
    scratch_types=[
        pltpu.VMEM((N_CHUNK, B_CHUNK), jnp.int32),
        pltpu.VMEM((N_CHUNK, B_CHUNK), jnp.int32),
        pltpu.VMEM((2, B_CHUNK, D_MODEL), jnp.float32),
        pltpu.SemaphoreType.DMA,
        pltpu.SemaphoreType.DMA,
    ],
    compiler_params=pltpu.CompilerParams(use_tc_tiling_on_sc=False),
)
def _gather(table_hbm, idx_hbm, out_hbm, idx_v, ridx_v, rows_v, gsem, wsem):
    wid = lax.axis_index("s") * NC + lax.axis_index("c")
    base = wid * B_PER_W
    pltpu.sync_copy(idx_hbm.at[wid], idx_v)

    for j in range(N_CHUNK):
        def body(k, carry):
            sl = pl.ds(k * LANES, LANES)
            e = idx_v[j, sl]
            ridx_v[j, sl] = ((e & (Q - 1)) << 2) | (e >> 18)
            return carry

        lax.fori_loop(0, B_CHUNK // LANES, body, 0)

    g_prev = pltpu.async_copy(table_hbm.at[ridx_v.at[0]], rows_v.at[0], gsem)
    for j in range(N_CHUNK):
        par = j % 2
        if j + 1 < N_CHUNK:
            g_next = pltpu.async_copy(
                table_hbm.at[ridx_v.at[j + 1]], rows_v.at[1 - par], gsem
            )
        g_prev.wait()
        pltpu.async_copy(
            rows_v.at[par],
            out_hbm.at[pl.ds(base + j * B_CHUNK, B_CHUNK)],
            wsem,
        ).wait()
        if j + 1 < N_CHUNK:
            g_prev = g_next


def kernel(x, W_E):
    idx = x.reshape(NW, N_CHUNK, B_CHUNK)
    table = _stack_t(W_E).reshape(4 * Q, D_MODEL)
    out = _gather(table, idx)
    return out.reshape(BATCH, HIST, D_MODEL)


# speedup vs baseline: 1.6530x; 1.1569x over previous
import functools

import jax
import jax.numpy as jnp
from jax import lax
from jax.experimental import pallas as pl
from jax.experimental.pallas import tpu as pltpu
from jax.experimental.pallas import tpu_sc as plsc

D_MODEL = 32
D_VOCAB = 1000000
BATCH = 16384
HIST = 20

NUM_IDX = BATCH * HIST

NC = 2
NS = 16
NW = NC * NS

B_PER_W = NUM_IDX // NW
N_CHUNK = 8
B_CHUNK = B_PER_W // N_CHUNK

LANES = 16


_VC = 8192
Q = 1 << 18
_NB = Q // _VC


def _stack_t_body(w0, w1, w2, w3, out_ref):
    a = jnp.concatenate([w0[...], w1[...], w2[...], w3[...]], axis=0)
    out_ref[...] = a.T


def _stack_t(w):
    qb = Q // _VC
    last = (D_VOCAB - 1) // _VC
    specs = [
        pl.BlockSpec(
            (D_MODEL, _VC),
            functools.partial(
                lambda q, i: (0, jnp.minimum(q * qb + i, last)), q
            ),
        )
        for q in range(4)
    ]
    return pl.pallas_call(
        _stack_t_body,
        grid=(_NB,),
        in_specs=specs,
        out_specs=pl.BlockSpec((_VC, 128), lambda i: (i, 0)),
        out_shape=jax.ShapeDtypeStruct((Q, 128), jnp.float32),
    )(w, w, w, w)


@functools.partial(
    pl.kernel,
    mesh=plsc.VectorSubcoreMesh(core_axis_name="c", subcore_axis_name="s"),
    out_type=jax.ShapeDtypeStruct((NUM_IDX, D_MODEL), jnp.float32),
    scratch_types=[
        pltpu.VMEM((N_CHUNK, B_CHUNK), jnp.int32),
        pltpu.VMEM((N_CHUNK, B_CHUNK), jnp.int32),
        pltpu.VMEM((2, B_CHUNK, D_MODEL), jnp.float32),
        pltpu.SemaphoreType.DMA,
        pltpu.SemaphoreType.DMA,
    ],
    compiler_params=pltpu.CompilerParams(use_tc_tiling_on_sc=False),
)
def _gather(table_hbm, idx_hbm, out_hbm, idx_v, ridx_v, rows_v, gsem, wsem):
    wid = lax.axis_index("s") * NC + lax.axis_index("c")
    base = wid * B_PER_W
    pltpu.sync_copy(idx_hbm.at[wid], idx_v)

    for j in range(N_CHUNK):
        def body(k, carry):
            sl = pl.ds(k * LANES, LANES)
            e = idx_v[j, sl]
            ridx_v[j, sl] = ((e & (Q - 1)) << 2) | (e >> 18)
            return carry

        lax.fori_loop(0, B_CHUNK // LANES, body, 0)

    g_prev = pltpu.async_copy(table_hbm.at[ridx_v.at[0]], rows_v.at[0], gsem)
    for j in range(N_CHUNK):
        par = j % 2
        if j + 1 < N_CHUNK:
            g_next = pltpu.async_copy(
                table_hbm.at[ridx_v.at[j + 1]], rows_v.at[1 - par], gsem
            )
        g_prev.wait()
        pltpu.async_copy(
            rows_v.at[par],
            out_hbm.at[pl.ds(base + j * B_CHUNK, B_CHUNK)],
            wsem,
        ).wait()
        if j + 1 < N_CHUNK:
            g_prev = g_next


def kernel(x, W_E):
    idx = x.reshape(NW, N_CHUNK, B_CHUNK)
    table = _stack_t(W_E).reshape(4 * Q, D_MODEL)
    out = _gather(table, idx)
    return out.reshape(BATCH, HIST, D_MODEL)


# speedup vs baseline: 1.6746x; 1.0131x over previous
import functools

import jax
import jax.numpy as jnp
from jax import lax
from jax.experimental import pallas as pl
from jax.experimental.pallas import tpu as pltpu
from jax.experimental.pallas import tpu_sc as plsc

D_MODEL = 32
D_VOCAB = 1000000
BATCH = 16384
HIST = 20

NUM_IDX = BATCH * HIST

NC = 2
NS = 16
NW = NC * NS

B_PER_W = NUM_IDX // NW
N_CHUNK = 8
B_CHUNK = B_PER_W // N_CHUNK

LANES = 16


_VC = 16384
Q = 1 << 18
_NB = Q // _VC


def _stack_t_body(w0, w1, w2, w3, out_ref):
    a = jnp.concatenate([w0[...], w1[...], w2[...], w3[...]], axis=0)
    out_ref[...] = a.T


def _stack_t(w):
    qb = Q // _VC
    last = (D_VOCAB - 1) // _VC
    specs = [
        pl.BlockSpec(
            (D_MODEL, _VC),
            functools.partial(
                lambda q, i: (0, jnp.minimum(q * qb + i, last)), q
            ),
        )
        for q in range(4)
    ]
    return pl.pallas_call(
        _stack_t_body,
        grid=(_NB,),
        in_specs=specs,
        out_specs=pl.BlockSpec((_VC, 128), lambda i: (i, 0)),
        out_shape=jax.ShapeDtypeStruct((Q, 128), jnp.float32),
    )(w, w, w, w)


@functools.partial(
    pl.kernel,
    mesh=plsc.VectorSubcoreMesh(core_axis_name="c", subcore_axis_name="s"),
    out_type=jax.ShapeDtypeStruct((NUM_IDX, D_MODEL), jnp.float32),
    scratch_types=[
        pltpu.VMEM((N_CHUNK, B_CHUNK), jnp.int32),
        pltpu.VMEM((N_CHUNK, B_CHUNK), jnp.int32),
        pltpu.VMEM((2, B_CHUNK, D_MODEL), jnp.float32),
        pltpu.SemaphoreType.DMA,
        pltpu.SemaphoreType.DMA,
    ],
    compiler_params=pltpu.CompilerParams(use_tc_tiling_on_sc=False),
)
def _gather(table_hbm, idx_hbm, out_hbm, idx_v, ridx_v, rows_v, gsem, wsem):
    wid = lax.axis_index("s") * NC + lax.axis_index("c")
    base = wid * B_PER_W
    pltpu.sync_copy(idx_hbm.at[wid], idx_v)

    for j in range(N_CHUNK):
        def body(k, carry):
            sl = pl.ds(k * LANES, LANES)
            e = idx_v[j, sl]
            ridx_v[j, sl] = ((e & (Q - 1)) << 2) | (e >> 18)
            return carry

        lax.fori_loop(0, B_CHUNK // LANES, body, 0)

    g_prev = pltpu.async_copy(table_hbm.at[ridx_v.at[0]], rows_v.at[0], gsem)
    for j in range(N_CHUNK):
        par = j % 2
        if j + 1 < N_CHUNK:
            g_next = pltpu.async_copy(
                table_hbm.at[ridx_v.at[j + 1]], rows_v.at[1 - par], gsem
            )
        g_prev.wait()
        pltpu.async_copy(
            rows_v.at[par],
            out_hbm.at[pl.ds(base + j * B_CHUNK, B_CHUNK)],
            wsem,
        ).wait()
        if j + 1 < N_CHUNK:
            g_prev = g_next


def kernel(x, W_E):
    idx = x.reshape(NW, N_CHUNK, B_CHUNK)
    table = _stack_t(W_E).reshape(4 * Q, D_MODEL)
    out = _gather(table, idx)
    return out.reshape(BATCH, HIST, D_MODEL)
